# baseline (speedup 1.0000x reference)
import functools

import jax
import jax.numpy as jnp
import numpy as np
from jax.experimental import pallas as pl
from jax.experimental.pallas import tpu as pltpu

D = 768
PS = 16
IMG = 224
NPATCH = (IMG // PS) ** 2
PD = PS * PS
NE = 8
NH = 8
DH = D // NH
HID = 256
B = 32
T = B * NPATCH
P = 28
HP = jax.lax.Precision.HIGHEST

SCALE = 1.0 / np.sqrt(DH)


_STD = (((1,), (0,)), ((), ()))
_TR = (((1,), (1,)), ((), ()))


def _dot(a, b):
    bh = b.astype(jnp.bfloat16)
    r1 = b - bh.astype(jnp.float32)
    bm = r1.astype(jnp.bfloat16)
    bl = (r1 - bm.astype(jnp.float32)).astype(jnp.bfloat16)
    d = lambda v: jax.lax.dot_general(
        a, v, _STD, preferred_element_type=jnp.float32)
    return d(bh) + d(bm) + d(bl)


def _split(a):
    ah = a.astype(jnp.bfloat16)
    return ah, (a - ah.astype(jnp.float32)).astype(jnp.bfloat16)


def _dotf(a, b):
    return _mix3(a, b, _STD)


def _attn_patch(q, k, v):
    q8 = jnp.concatenate([q[:, h * DH:(h + 1) * DH] for h in range(NH)], 0)
    k8 = jnp.concatenate([k[:, h * DH:(h + 1) * DH] for h in range(NH)], 0)
    v8 = jnp.concatenate([v[:, h * DH:(h + 1) * DH] for h in range(NH)], 0)
    s = _mix3(q8, k8, _TR) * SCALE
    n = NH * B
    blk = (jax.lax.broadcasted_iota(jnp.int32, (n, n), 0) // B ==
           jax.lax.broadcasted_iota(jnp.int32, (n, n), 1) // B)
    s = jnp.where(blk, s, -1e30)
    m = jnp.max(s, axis=-1, keepdims=True)
    e = jnp.exp(s - m)
    pa = e / jnp.sum(e, axis=-1, keepdims=True)
    o8 = _mix3(pa, v8, _STD)
    return jnp.concatenate([o8[h * B:(h + 1) * B, :] for h in range(NH)], 1)


def _mix3(a, b, dims):
    bh = b.astype(jnp.bfloat16)
    r1 = b - bh.astype(jnp.float32)
    bm = r1.astype(jnp.bfloat16)
    bl = (r1 - bm.astype(jnp.float32)).astype(jnp.bfloat16)
    d = lambda v: jax.lax.dot_general(
        a, v, dims, preferred_element_type=jnp.float32)
    return d(bh) + d(bm) + d(bl)


def _gate_weights(logits):
    m = jnp.max(logits, axis=-1, keepdims=True)
    e = jnp.exp(logits - m)
    p = e / jnp.sum(e, axis=-1, keepdims=True)
    idx = jax.lax.broadcasted_iota(jnp.int32, p.shape, 1)
    p1 = jnp.max(p, axis=-1, keepdims=True)
    i1 = jnp.min(jnp.where(p == p1, idx, NE), axis=-1, keepdims=True)
    pm = jnp.where(idx == i1, -jnp.inf, p)
    p2 = jnp.max(pm, axis=-1, keepdims=True)
    i2 = jnp.min(jnp.where(pm == p2, idx, NE), axis=-1, keepdims=True)
    return jnp.where((idx == i1) | (idx == i2), p, 0.0) / (p1 + p2)


def _kx_body_common(x2, refs, qkv_s, xo_ref, wi_ref):
    (inwt, inb, qwh, qwm, qwl, qkvb, owt, ob, gwt, gb) = refs
    rows = x2.shape[0]
    xi = _dot(x2, inwt[...]) + inb[...]
    dq = lambda v: jax.lax.dot_general(
        xi, v, _STD, preferred_element_type=jnp.float32)
    qkv = dq(qwh[...]) + dq(qwm[...]) + dq(qwl[...]) + qkvb[...]
    qkv_s[...] = qkv.reshape(B, P, 3 * D)

    def attn_step(p_, _):
        xp3 = qkv_s[:, pl.ds(p_, 1), :].reshape(B, 3 * D)
        o = _attn_patch(xp3[:, :D], xp3[:, D:2 * D], xp3[:, 2 * D:])
        qkv_s[:, pl.ds(p_, 1), 0:D] = o.reshape(B, 1, D)
        return 0

    jax.lax.fori_loop(0, P, attn_step, 0, unroll=7)
    xo = _dot(qkv_s[:, :, 0:D].reshape(rows, D), owt[...]) + ob[...]
    xo_ref[...] = xo.reshape(B, 1, P, D)
    wi = _gate_weights(_dot(xo, gwt[...]) + gb[...])
    wi_ref[...] = wi.reshape(B, 1, P, NE)


def _kx1_body(xp_ref, pos_ref, pwt_ref, *rest):
    refs = rest[:10]
    xo_ref, wi_ref, qkv_s = rest[10:]
    x = xp_ref[...].reshape(B * P, PD)
    x2 = _dot(x, pwt_ref[...])
    x2 = (x2.reshape(B, P, D) + pos_ref[0][None]).reshape(B * P, D)
    _kx_body_common(x2, refs, qkv_s, xo_ref, wi_ref)


def _kx2_body(xin_ref, *rest):
    refs = rest[:10]
    xo_ref, wi_ref, qkv_s = rest[10:]
    x2 = xin_ref[...].reshape(B * P, D)
    _kx_body_common(x2, refs, qkv_s, xo_ref, wi_ref)


def _make_ky_body(fast):
    edot = _dotf if fast else _dot

    def body(xo_ref, wi_ref, w1t, b1, w2t, b2, vwt, vb, cwt, cb,
             fv_ref, cls_ref):
        xo = xo_ref[...].reshape(B * P, D)
        wi = wi_ref[...].reshape(B * P, NE)
        rows = B * P

        def exp_step(i, acc):
            w1 = w1t[pl.ds(i, 1)].reshape(D, HID)
            h = jnp.maximum(edot(xo, w1) + b1[pl.ds(i, 1)].reshape(1, HID),
                            0.0)
            w2 = w2t[pl.ds(i, 1)].reshape(HID, D)
            eo = edot(h, w2) + b2[pl.ds(i, 1)].reshape(1, D)
            eidx = jax.lax.broadcasted_iota(jnp.int32, wi.shape, 1)
            wsel = jnp.sum(jnp.where(eidx == i, wi, 0.0), axis=1,
                           keepdims=True)
            return acc + eo * wsel

        acc = jax.lax.fori_loop(0, NE, exp_step,
                                jnp.zeros((rows, D), jnp.float32), unroll=8)
        fv = edot(acc, vwt[...]) + vb[...]
        cls = _dotf(acc, cwt[...]) + cb[...]
        fv_ref[...] = fv.reshape(B, 1, P, D)
        cls_ref[...] = cls.reshape(B, 1, P, D)

    return body


_ky_hi = _make_ky_body(False)
_ky_fast = _make_ky_body(True)

NG = NPATCH // P


def _blk(last):
    return pl.BlockSpec((B, 1, P, last), lambda i: (0, i, 0, 0))


def _c2(shp):
    return pl.BlockSpec(shp, lambda i: (0, 0))


def _c3(shp):
    return pl.BlockSpec(shp, lambda i: (0, 0, 0))


def _x_specs():
    return [_c2((D, D)), _c2((1, D)),
            _c2((D, 3 * D)), _c2((D, 3 * D)), _c2((D, 3 * D)),
            _c2((1, 3 * D)),
            _c2((D, D)), _c2((1, D)),
            _c2((D, NE)), _c2((1, NE))]


def _split3(w):
    wh = w.astype(jnp.bfloat16)
    r1 = w - wh.astype(jnp.float32)
    wm = r1.astype(jnp.bfloat16)
    wl = (r1 - wm.astype(jnp.float32)).astype(jnp.bfloat16)
    return wh, wm, wl


def _x_args(mp):
    qwh, qwm, qwl = _split3(mp["qkvW"].T)
    return (mp["inW"].T, mp["inb"].reshape(1, D),
            qwh, qwm, qwl, mp["qkvb"].reshape(1, 3 * D),
            mp["oW"].T, mp["ob"].reshape(1, D),
            mp["gW"].T, mp["gb"].reshape(1, NE))


def _f32s(shape):
    return jax.ShapeDtypeStruct(shape, jnp.float32)


def _x_layer(xin, posb, pwt, mp, first_layer):
    outs = [_f32s((B, NG, P, D)), _f32s((B, NG, P, NE))]
    ospec = [_blk(D), _blk(NE)]
    scratch = [pltpu.VMEM((B, P, 3 * D), jnp.float32)]
    if first_layer:
        return pl.pallas_call(
            _kx1_body, grid=(NG,),
            in_specs=[_blk(PD),
                      pl.BlockSpec((1, P, D), lambda i: (i, 0, 0)),
                      _c2((PD, D))] + _x_specs(),
            out_specs=ospec, out_shape=outs, scratch_shapes=scratch,
        )(xin, posb, pwt, *_x_args(mp))
    return pl.pallas_call(
        _kx2_body, grid=(NG,),
        in_specs=[_blk(D)] + _x_specs(),
        out_specs=ospec, out_shape=outs, scratch_shapes=scratch,
    )(xin, *_x_args(mp))


def _y_layer(xo, wi, mp, vWt, vb, cWt, cb, fast):
    return pl.pallas_call(
        _ky_fast if fast else _ky_hi, grid=(NG,),
        in_specs=[_blk(D), _blk(NE),
                  _c3((NE, D, HID)), _c3((NE, 1, HID)),
                  _c3((NE, HID, D)), _c3((NE, 1, D)),
                  _c2((D, D)), _c2((1, D)), _c2((D, D)), _c2((1, D))],
        out_specs=[_blk(D), _blk(D)],
        out_shape=[_f32s((B, NG, P, D)), _f32s((B, NG, P, D))],
    )(xo, wi, mp["W1"].transpose(0, 2, 1), mp["b1"].reshape(NE, 1, HID),
      mp["W2"].transpose(0, 2, 1), mp["b2"].reshape(NE, 1, D),
      vWt, vb, cWt, cb)


def kernel(x, params):
    n = IMG // PS
    xp4 = (x.reshape(B, n, PS, n, PS)
            .transpose(0, 1, 3, 2, 4)
            .reshape(B, NG, P, PD))
    pos = (params["pos"].reshape(NPATCH, D) +
           params["pb"].reshape(1, D)).reshape(NG, P, D)
    vWt = params["vW"].T
    vb = params["vb"].reshape(1, D)
    cWt = params["cW"].T
    cb = params["cb"].reshape(1, D)

    xo1, wi1 = _x_layer(xp4, pos, params["pW"].T, params["moe1"], True)
    fv1, cls1 = _y_layer(xo1, wi1, params["moe1"], vWt, vb, cWt, cb, False)
    xo2, wi2 = _x_layer(fv1, None, None, params["moe2"], False)
    fv2, cls2 = _y_layer(xo2, wi2, params["moe2"], vWt, vb, cWt, cb, True)

    sh = (B, NPATCH, D)
    return (fv1.reshape(sh), fv2.reshape(sh),
            cls1.reshape(sh), cls2.reshape(sh))


# speedup vs baseline: 1.0350x; 1.0350x over previous
import functools

import jax
import jax.numpy as jnp
import numpy as np
from jax.experimental import pallas as pl
from jax.experimental.pallas import tpu as pltpu

D = 768
PS = 16
IMG = 224
NPATCH = (IMG // PS) ** 2
PD = PS * PS
NE = 8
NH = 8
DH = D // NH
HID = 256
B = 32
T = B * NPATCH
P = 14
HP = jax.lax.Precision.HIGHEST

SCALE = 1.0 / np.sqrt(DH)


_STD = (((1,), (0,)), ((), ()))
_TR = (((1,), (1,)), ((), ()))


def _dot(a, b):
    bh = b.astype(jnp.bfloat16)
    r1 = b - bh.astype(jnp.float32)
    bm = r1.astype(jnp.bfloat16)
    bl = (r1 - bm.astype(jnp.float32)).astype(jnp.bfloat16)
    d = lambda v: jax.lax.dot_general(
        a, v, _STD, preferred_element_type=jnp.float32)
    return d(bh) + d(bm) + d(bl)


def _split(a):
    ah = a.astype(jnp.bfloat16)
    return ah, (a - ah.astype(jnp.float32)).astype(jnp.bfloat16)


def _dotf(a, b):
    ah, al = _split(a)
    bh, bl = _split(b)
    d = lambda u, v: jax.lax.dot_general(
        u, v, _STD, preferred_element_type=jnp.float32)
    return d(ah, bh) + d(ah, bl) + d(al, bh)


def _attn_patch(q, k, v):
    q8 = jnp.concatenate([q[:, h * DH:(h + 1) * DH] for h in range(NH)], 0)
    k8 = jnp.concatenate([k[:, h * DH:(h + 1) * DH] for h in range(NH)], 0)
    v8 = jnp.concatenate([v[:, h * DH:(h + 1) * DH] for h in range(NH)], 0)
    s = _mix3(q8, k8, _TR) * SCALE
    n = NH * B
    blk = (jax.lax.broadcasted_iota(jnp.int32, (n, n), 0) // B ==
           jax.lax.broadcasted_iota(jnp.int32, (n, n), 1) // B)
    s = jnp.where(blk, s, -1e30)
    m = jnp.max(s, axis=-1, keepdims=True)
    e = jnp.exp(s - m)
    pa = e / jnp.sum(e, axis=-1, keepdims=True)
    o8 = _mix3(pa, v8, _STD)
    return jnp.concatenate([o8[h * B:(h + 1) * B, :] for h in range(NH)], 1)


def _mix3(a, b, dims):
    bh = b.astype(jnp.bfloat16)
    r1 = b - bh.astype(jnp.float32)
    bm = r1.astype(jnp.bfloat16)
    bl = (r1 - bm.astype(jnp.float32)).astype(jnp.bfloat16)
    d = lambda v: jax.lax.dot_general(
        a, v, dims, preferred_element_type=jnp.float32)
    return d(bh) + d(bm) + d(bl)


def _gate_weights(logits):
    m = jnp.max(logits, axis=-1, keepdims=True)
    e = jnp.exp(logits - m)
    p = e / jnp.sum(e, axis=-1, keepdims=True)
    idx = jax.lax.broadcasted_iota(jnp.int32, p.shape, 1)
    p1 = jnp.max(p, axis=-1, keepdims=True)
    i1 = jnp.min(jnp.where(p == p1, idx, NE), axis=-1, keepdims=True)
    pm = jnp.where(idx == i1, -jnp.inf, p)
    p2 = jnp.max(pm, axis=-1, keepdims=True)
    i2 = jnp.min(jnp.where(pm == p2, idx, NE), axis=-1, keepdims=True)
    return jnp.where((idx == i1) | (idx == i2), p, 0.0) / (p1 + p2)


def _moe_body(x2, refs, qkv_s, fast_tail):
    (inwt, inb, qwh, qwm, qwl, qkvb, owt, ob, gwt, gb, w1t, b1, w2t, b2,
     vwt, vb, cwt, cb) = refs
    rows = x2.shape[0]
    xi = _dot(x2, inwt[...]) + inb[...]
    dq = lambda v: jax.lax.dot_general(
        xi, v, _STD, preferred_element_type=jnp.float32)
    qkv = dq(qwh[...]) + dq(qwm[...]) + dq(qwl[...]) + qkvb[...]
    qkv_s[...] = qkv.reshape(B, P, 3 * D)

    def attn_step(p_, _):
        xp3 = qkv_s[:, pl.ds(p_, 1), :].reshape(B, 3 * D)
        o = _attn_patch(xp3[:, :D], xp3[:, D:2 * D], xp3[:, 2 * D:])
        qkv_s[:, pl.ds(p_, 1), 0:D] = o.reshape(B, 1, D)
        return 0

    jax.lax.fori_loop(0, P, attn_step, 0, unroll=7)
    xo = _dot(qkv_s[:, :, 0:D].reshape(rows, D), owt[...]) + ob[...]
    wi = _gate_weights(_dot(xo, gwt[...]) + gb[...])
    edot = _dotf if fast_tail else _dot

    def exp_step(i, acc):
        w1 = w1t[pl.ds(i, 1)].reshape(D, HID)
        h = jnp.maximum(edot(xo, w1) + b1[pl.ds(i, 1)].reshape(1, HID), 0.0)
        w2 = w2t[pl.ds(i, 1)].reshape(HID, D)
        eo = edot(h, w2) + b2[pl.ds(i, 1)].reshape(1, D)
        eidx = jax.lax.broadcasted_iota(jnp.int32, wi.shape, 1)
        wsel = jnp.sum(jnp.where(eidx == i, wi, 0.0), axis=1, keepdims=True)
        return acc + eo * wsel

    acc = jax.lax.fori_loop(0, NE, exp_step,
                            jnp.zeros((rows, D), jnp.float32), unroll=8)
    fv = (edot(acc, vwt[...]) + vb[...])
    cls = _dotf(acc, cwt[...]) + cb[...]
    return fv.reshape(B, P, D), cls.reshape(B, P, D)


def _k1_body(xp_ref, pos_ref, pwt_ref, *rest):
    refs = rest[:18]
    fv_ref, cls_ref, qkv_s = rest[18:]
    x = xp_ref[...].reshape(B * P, PD)
    x2 = _dot(x, pwt_ref[...])
    x2 = (x2.reshape(B, P, D) + pos_ref[0][None]).reshape(B * P, D)
    fv, cls = _moe_body(x2, refs, qkv_s, False)
    fv_ref[...] = fv.reshape(B, 1, P, D)
    cls_ref[...] = cls.reshape(B, 1, P, D)


def _k2_body(xin_ref, *rest):
    refs = rest[:18]
    fv_ref, cls_ref, qkv_s = rest[18:]
    x2 = xin_ref[...].reshape(B * P, D)
    fv, cls = _moe_body(x2, refs, qkv_s, True)
    fv_ref[...] = fv.reshape(B, 1, P, D)
    cls_ref[...] = cls.reshape(B, 1, P, D)


def _split3(w):
    wh = w.astype(jnp.bfloat16)
    r1 = w - wh.astype(jnp.float32)
    wm = r1.astype(jnp.bfloat16)
    wl = (r1 - wm.astype(jnp.float32)).astype(jnp.bfloat16)
    return wh, wm, wl


def _moe_args(mp, vWt, vb, cWt, cb):
    qwh, qwm, qwl = _split3(mp["qkvW"].T)
    return (
        mp["inW"].T, mp["inb"].reshape(1, D),
        qwh, qwm, qwl, mp["qkvb"].reshape(1, 3 * D),
        mp["oW"].T, mp["ob"].reshape(1, D),
        mp["gW"].T, mp["gb"].reshape(1, NE),
        mp["W1"].transpose(0, 2, 1), mp["b1"].reshape(NE, 1, HID),
        mp["W2"].transpose(0, 2, 1), mp["b2"].reshape(NE, 1, D),
        vWt, vb, cWt, cb,
    )


def _moe_specs():
    c2 = lambda shp: pl.BlockSpec(shp, lambda i: (0, 0))
    c3 = lambda shp: pl.BlockSpec(shp, lambda i: (0, 0, 0))
    return [
        c2((D, D)), c2((1, D)),
        c2((D, 3 * D)), c2((D, 3 * D)), c2((D, 3 * D)), c2((1, 3 * D)),
        c2((D, D)), c2((1, D)),
        c2((D, NE)), c2((1, NE)),
        c3((NE, D, HID)), c3((NE, 1, HID)),
        c3((NE, HID, D)), c3((NE, 1, D)),
        c2((D, D)), c2((1, D)), c2((D, D)), c2((1, D)),
    ]


NG = NPATCH // P


def _blk(last):
    return pl.BlockSpec((B, 1, P, last), lambda i: (0, i, 0, 0))


def kernel(x, params):
    n = IMG // PS
    xp4 = (x.reshape(B, n, PS, n, PS)
            .transpose(0, 1, 3, 2, 4)
            .reshape(B, NG, P, PD))
    pos = (params["pos"].reshape(NPATCH, D) +
           params["pb"].reshape(1, D)).reshape(NG, P, D)
    vWt = params["vW"].T
    vb = params["vb"].reshape(1, D)
    cWt = params["cW"].T
    cb = params["cb"].reshape(1, D)
    out_sh = jax.ShapeDtypeStruct((B, NG, P, D), jnp.float32)

    fv1, cls1 = pl.pallas_call(
        _k1_body,
        grid=(NG,),
        in_specs=[_blk(PD),
                  pl.BlockSpec((1, P, D), lambda i: (i, 0, 0)),
                  pl.BlockSpec((PD, D), lambda i: (0, 0))] + _moe_specs(),
        out_specs=[_blk(D), _blk(D)],
        out_shape=[out_sh, out_sh],
        scratch_shapes=[pltpu.VMEM((B, P, 3 * D), jnp.float32)],
    )(xp4, pos, params["pW"].T, *_moe_args(params["moe1"], vWt, vb, cWt, cb))

    fv2, cls2 = pl.pallas_call(
        _k2_body,
        grid=(NG,),
        in_specs=[_blk(D)] + _moe_specs(),
        out_specs=[_blk(D), _blk(D)],
        out_shape=[out_sh, out_sh],
        scratch_shapes=[pltpu.VMEM((B, P, 3 * D), jnp.float32)],
    )(fv1, *_moe_args(params["moe2"], vWt, vb, cWt, cb))

    sh = (B, NPATCH, D)
    return (fv1.reshape(sh), fv2.reshape(sh),
            cls1.reshape(sh), cls2.reshape(sh))
